# baseline (speedup 1.0000x reference)
import jax
import jax.numpy as jnp
from jax import lax
from jax.experimental import pallas as pl
from jax.experimental.pallas import tpu as pltpu
from jax.experimental.pallas import tpu_sc as plsc

EMB = 64
NC, NS = 2, 16
NW = NC * NS
CT_BLK = 4096
NBC = 8
LPAD = 56


def _ct_body(tt_ref, w_ref, ct_ref):
    ct_ref[...] = lax.dot_general(
        tt_ref[...], w_ref[...], (((0,), (0,)), ((), ())),
        preferred_element_type=jnp.float32,
    )


def _tc_combined_table(table_t, wct):
    v = table_t.shape[1]
    nblk = -(-v // CT_BLK)
    return pl.pallas_call(
        _ct_body,
        grid=(nblk,),
        in_specs=[
            pl.BlockSpec((EMB, CT_BLK), lambda i: (0, i)),
            pl.BlockSpec((EMB, 2 * EMB), lambda i: (0, 0)),
        ],
        out_specs=pl.BlockSpec((CT_BLK, 2 * EMB), lambda i: (i, 0)),
        out_shape=jax.ShapeDtypeStruct((nblk * CT_BLK, 2 * EMB), jnp.float32),
    )(table_t, wct)


def _chunk(ct_hbm, x_hbm, big_hbm, idx_v, crows_v, sem, bj):
    pltpu.sync_copy(x_hbm.at[pl.ds(bj, NBC)], idx_v)
    return [
        pltpu.async_copy(
            ct_hbm.at[idx_v.at[i]], crows_v.at[i, pl.ds(0, x_hbm.shape[1])],
            sem,
        )
        for i in range(NBC)
    ]


def _gather_body(ct_hbm, x_hbm, big_hbm, idx_a, idx_b, crows_a, crows_b,
                 sem_a, sem_b):
    wid = lax.axis_index("s") * NC + lax.axis_index("c")
    nb_per_w = x_hbm.shape[0] // NW
    b0 = wid * nb_per_w
    n_pairs = nb_per_w // (2 * NBC)

    def body(p, carry):
        bja = b0 + p * 2 * NBC
        bjb = bja + NBC
        starts_a = _chunk(ct_hbm, x_hbm, big_hbm, idx_a, crows_a, sem_a, bja)
        starts_b = _chunk(ct_hbm, x_hbm, big_hbm, idx_b, crows_b, sem_b, bjb)
        for s in starts_a:
            s.wait()
        pltpu.sync_copy(crows_a, big_hbm.at[pl.ds(bja, NBC)])
        for s in starts_b:
            s.wait()
        pltpu.sync_copy(crows_b, big_hbm.at[pl.ds(bjb, NBC)])
        return carry

    lax.fori_loop(0, n_pairs, body, 0)


def _sc_gather2(ct, x):
    b, l = x.shape
    mesh = plsc.VectorSubcoreMesh(core_axis_name="c", subcore_axis_name="s")
    k = pl.kernel(
        _gather_body,
        out_type=jax.ShapeDtypeStruct((b, LPAD, 2 * EMB), jnp.float32),
        mesh=mesh,
        scratch_types=[
            pltpu.VMEM((NBC, l), jnp.int32),
            pltpu.VMEM((NBC, l), jnp.int32),
            pltpu.VMEM((NBC, LPAD, 2 * EMB), jnp.float32),
            pltpu.VMEM((NBC, LPAD, 2 * EMB), jnp.float32),
            pltpu.SemaphoreType.DMA,
            pltpu.SemaphoreType.DMA,
        ],
        compiler_params=pltpu.CompilerParams(use_tc_tiling_on_sc=False),
    )
    return k(ct, x)


NBF = 64


def _fin_body(half, big_ref, t_ref):
    v = big_ref[...]
    l = t_ref.shape[2]
    sl = v[:, :l, half * EMB:(half + 1) * EMB]
    t_ref[...] = jnp.transpose(sl, (0, 2, 1))


def _tc_finish_half(big, l, half):
    b = big.shape[0]
    return pl.pallas_call(
        lambda big_ref, t_ref: _fin_body(half, big_ref, t_ref),
        grid=(b // NBF,),
        in_specs=[
            pl.BlockSpec((NBF, LPAD, 2 * EMB), lambda i: (i, 0, 0))
        ],
        out_specs=pl.BlockSpec((NBF, EMB, l), lambda i: (i, 0, 0)),
        out_shape=jax.ShapeDtypeStruct((b, EMB, l), jnp.float32),
    )(big)


def kernel(x, embed_table, router_w):
    l = x.shape[1]
    wct = jnp.concatenate(
        [jnp.eye(EMB, dtype=jnp.float32), router_w.T], axis=1
    )
    ct = _tc_combined_table(embed_table.T, wct)
    big = _sc_gather2(ct, x.astype(jnp.int32))
    ht = _tc_finish_half(big, l, 0)
    lt = _tc_finish_half(big, l, 1)
    return (jnp.transpose(ht, (0, 2, 1)), jnp.transpose(lt, (0, 2, 1)))


# speedup vs baseline: 1.1846x; 1.1846x over previous
import jax
import jax.numpy as jnp
from jax import lax
from jax.experimental import pallas as pl
from jax.experimental.pallas import tpu as pltpu
from jax.experimental.pallas import tpu_sc as plsc

EMB = 64
NC, NS = 2, 16
NW = NC * NS
CT_BLK = 8192
NBC = 8
LPAD = 56


def _ct_body(tt_ref, w_ref, ct_ref):
    ct_ref[...] = lax.dot_general(
        tt_ref[...], w_ref[...], (((0,), (0,)), ((), ())),
        preferred_element_type=jnp.float32,
    )


def _tc_combined_table(table_t, wct):
    v = table_t.shape[1]
    nblk = -(-v // CT_BLK)
    return pl.pallas_call(
        _ct_body,
        grid=(nblk,),
        in_specs=[
            pl.BlockSpec((EMB, CT_BLK), lambda i: (0, i)),
            pl.BlockSpec((EMB, 2 * EMB), lambda i: (0, 0)),
        ],
        out_specs=pl.BlockSpec((CT_BLK, 2 * EMB), lambda i: (i, 0)),
        out_shape=jax.ShapeDtypeStruct((nblk * CT_BLK, 2 * EMB), jnp.float32),
    )(table_t, wct)


def _chunk(ct_hbm, x_hbm, big_hbm, idx_v, crows_v, sem, bj):
    pltpu.sync_copy(x_hbm.at[pl.ds(bj, NBC)], idx_v)
    return [
        pltpu.async_copy(
            ct_hbm.at[idx_v.at[i]], crows_v.at[i, pl.ds(0, x_hbm.shape[1])],
            sem,
        )
        for i in range(NBC)
    ]


def _gather_body(ct_hbm, x_hbm, big_hbm, idx_a, idx_b, crows_a, crows_b,
                 sem_a, sem_b):
    wid = lax.axis_index("s") * NC + lax.axis_index("c")
    nb_per_w = x_hbm.shape[0] // NW
    b0 = wid * nb_per_w
    n_pairs = nb_per_w // (2 * NBC)

    def body(p, carry):
        bja = b0 + p * 2 * NBC
        bjb = bja + NBC
        starts_a = _chunk(ct_hbm, x_hbm, big_hbm, idx_a, crows_a, sem_a, bja)
        starts_b = _chunk(ct_hbm, x_hbm, big_hbm, idx_b, crows_b, sem_b, bjb)
        for s in starts_a:
            s.wait()
        pltpu.sync_copy(crows_a, big_hbm.at[pl.ds(bja, NBC)])
        for s in starts_b:
            s.wait()
        pltpu.sync_copy(crows_b, big_hbm.at[pl.ds(bjb, NBC)])
        return carry

    lax.fori_loop(0, n_pairs, body, 0)


def _sc_gather2(ct, x):
    b, l = x.shape
    mesh = plsc.VectorSubcoreMesh(core_axis_name="c", subcore_axis_name="s")
    k = pl.kernel(
        _gather_body,
        out_type=jax.ShapeDtypeStruct((b, LPAD, 2 * EMB), jnp.float32),
        mesh=mesh,
        scratch_types=[
            pltpu.VMEM((NBC, l), jnp.int32),
            pltpu.VMEM((NBC, l), jnp.int32),
            pltpu.VMEM((NBC, LPAD, 2 * EMB), jnp.float32),
            pltpu.VMEM((NBC, LPAD, 2 * EMB), jnp.float32),
            pltpu.SemaphoreType.DMA,
            pltpu.SemaphoreType.DMA,
        ],
        compiler_params=pltpu.CompilerParams(use_tc_tiling_on_sc=False),
    )
    return k(ct, x)


NBF = 128


def _fin_body(big_ref, ht_ref, lt_ref):
    v = big_ref[...]
    l = ht_ref.shape[2]
    ht_ref[...] = jnp.transpose(v[:, :l, :EMB], (0, 2, 1))
    lt_ref[...] = jnp.transpose(v[:, :l, EMB:], (0, 2, 1))


def _tc_finish(big, l):
    b = big.shape[0]
    return pl.pallas_call(
        _fin_body,
        grid=(b // NBF,),
        in_specs=[pl.BlockSpec((NBF, LPAD, 2 * EMB), lambda i: (i, 0, 0))],
        out_specs=[
            pl.BlockSpec((NBF, EMB, l), lambda i: (i, 0, 0)),
            pl.BlockSpec((NBF, EMB, l), lambda i: (i, 0, 0)),
        ],
        out_shape=[
            jax.ShapeDtypeStruct((b, EMB, l), jnp.float32),
            jax.ShapeDtypeStruct((b, EMB, l), jnp.float32),
        ],
    )(big)


def kernel(x, embed_table, router_w):
    l = x.shape[1]
    wct = jnp.concatenate(
        [jnp.eye(EMB, dtype=jnp.float32), router_w.T], axis=1
    )
    ct = _tc_combined_table(embed_table.T, wct)
    big = _sc_gather2(ct, x.astype(jnp.int32))
    ht, lt = _tc_finish(big, l)
    return (jnp.transpose(ht, (0, 2, 1)), jnp.transpose(lt, (0, 2, 1)))


# speedup vs baseline: 1.2219x; 1.0315x over previous
import jax
import jax.numpy as jnp
from jax import lax
from jax.experimental import pallas as pl
from jax.experimental.pallas import tpu as pltpu
from jax.experimental.pallas import tpu_sc as plsc

EMB = 64
NC, NS = 2, 16
NW = NC * NS
CT_BLK = 16384
NBC = 8
LPAD = 56


def _ct_body(tt_ref, w_ref, ct_ref):
    ct_ref[...] = lax.dot_general(
        tt_ref[...], w_ref[...], (((0,), (0,)), ((), ())),
        preferred_element_type=jnp.float32,
    )


def _tc_combined_table(table_t, wct):
    v = table_t.shape[1]
    nblk = -(-v // CT_BLK)
    return pl.pallas_call(
        _ct_body,
        grid=(nblk,),
        in_specs=[
            pl.BlockSpec((EMB, CT_BLK), lambda i: (0, i)),
            pl.BlockSpec((EMB, 2 * EMB), lambda i: (0, 0)),
        ],
        out_specs=pl.BlockSpec((CT_BLK, 2 * EMB), lambda i: (i, 0)),
        out_shape=jax.ShapeDtypeStruct((nblk * CT_BLK, 2 * EMB), jnp.float32),
    )(table_t, wct)


def _chunk(ct_hbm, x_hbm, big_hbm, idx_v, crows_v, sem, bj):
    pltpu.sync_copy(x_hbm.at[pl.ds(bj, NBC)], idx_v)
    return [
        pltpu.async_copy(
            ct_hbm.at[idx_v.at[i]], crows_v.at[i, pl.ds(0, x_hbm.shape[1])],
            sem,
        )
        for i in range(NBC)
    ]


def _gather_body(ct_hbm, x_hbm, big_hbm, idx_a, idx_b, crows_a, crows_b,
                 sem_a, sem_b):
    wid = lax.axis_index("s") * NC + lax.axis_index("c")
    nb_per_w = x_hbm.shape[0] // NW
    b0 = wid * nb_per_w
    n_pairs = nb_per_w // (2 * NBC)

    def body(p, carry):
        bja = b0 + p * 2 * NBC
        bjb = bja + NBC
        starts_a = _chunk(ct_hbm, x_hbm, big_hbm, idx_a, crows_a, sem_a, bja)
        starts_b = _chunk(ct_hbm, x_hbm, big_hbm, idx_b, crows_b, sem_b, bjb)
        for s in starts_a:
            s.wait()
        pltpu.sync_copy(crows_a, big_hbm.at[pl.ds(bja, NBC)])
        for s in starts_b:
            s.wait()
        pltpu.sync_copy(crows_b, big_hbm.at[pl.ds(bjb, NBC)])
        return carry

    lax.fori_loop(0, n_pairs, body, 0)


def _sc_gather2(ct, x):
    b, l = x.shape
    mesh = plsc.VectorSubcoreMesh(core_axis_name="c", subcore_axis_name="s")
    k = pl.kernel(
        _gather_body,
        out_type=jax.ShapeDtypeStruct((b, LPAD, 2 * EMB), jnp.float32),
        mesh=mesh,
        scratch_types=[
            pltpu.VMEM((NBC, l), jnp.int32),
            pltpu.VMEM((NBC, l), jnp.int32),
            pltpu.VMEM((NBC, LPAD, 2 * EMB), jnp.float32),
            pltpu.VMEM((NBC, LPAD, 2 * EMB), jnp.float32),
            pltpu.SemaphoreType.DMA,
            pltpu.SemaphoreType.DMA,
        ],
        compiler_params=pltpu.CompilerParams(use_tc_tiling_on_sc=False),
    )
    return k(ct, x)


NBF = 256


def _fin_body(big_ref, ht_ref, lt_ref):
    v = big_ref[...]
    l = ht_ref.shape[2]
    ht_ref[...] = jnp.transpose(v[:, :l, :EMB], (0, 2, 1))
    lt_ref[...] = jnp.transpose(v[:, :l, EMB:], (0, 2, 1))


def _tc_finish(big, l):
    b = big.shape[0]
    return pl.pallas_call(
        _fin_body,
        grid=(b // NBF,),
        in_specs=[pl.BlockSpec((NBF, LPAD, 2 * EMB), lambda i: (i, 0, 0))],
        out_specs=[
            pl.BlockSpec((NBF, EMB, l), lambda i: (i, 0, 0)),
            pl.BlockSpec((NBF, EMB, l), lambda i: (i, 0, 0)),
        ],
        out_shape=[
            jax.ShapeDtypeStruct((b, EMB, l), jnp.float32),
            jax.ShapeDtypeStruct((b, EMB, l), jnp.float32),
        ],
    )(big)


def kernel(x, embed_table, router_w):
    l = x.shape[1]
    wct = jnp.concatenate(
        [jnp.eye(EMB, dtype=jnp.float32), router_w.T], axis=1
    )
    ct = _tc_combined_table(embed_table.T, wct)
    big = _sc_gather2(ct, x.astype(jnp.int32))
    ht, lt = _tc_finish(big, l)
    return (jnp.transpose(ht, (0, 2, 1)), jnp.transpose(lt, (0, 2, 1)))


# speedup vs baseline: 1.2261x; 1.0035x over previous
import jax
import jax.numpy as jnp
from jax import lax
from jax.experimental import pallas as pl
from jax.experimental.pallas import tpu as pltpu
from jax.experimental.pallas import tpu_sc as plsc

EMB = 64
NC, NS = 2, 16
NW = NC * NS
CT_BLK = 32768
NBC = 8
LPAD = 56


def _ct_body(tt_ref, w_ref, ct_ref):
    ct_ref[...] = lax.dot_general(
        tt_ref[...], w_ref[...], (((0,), (0,)), ((), ())),
        preferred_element_type=jnp.float32,
    )


def _tc_combined_table(table_t, wct):
    v = table_t.shape[1]
    nblk = -(-v // CT_BLK)
    return pl.pallas_call(
        _ct_body,
        grid=(nblk,),
        in_specs=[
            pl.BlockSpec((EMB, CT_BLK), lambda i: (0, i)),
            pl.BlockSpec((EMB, 2 * EMB), lambda i: (0, 0)),
        ],
        out_specs=pl.BlockSpec((CT_BLK, 2 * EMB), lambda i: (i, 0)),
        out_shape=jax.ShapeDtypeStruct((nblk * CT_BLK, 2 * EMB), jnp.float32),
    )(table_t, wct)


def _chunk(ct_hbm, x_hbm, big_hbm, idx_v, crows_v, sem, bj):
    pltpu.sync_copy(x_hbm.at[pl.ds(bj, NBC)], idx_v)
    return [
        pltpu.async_copy(
            ct_hbm.at[idx_v.at[i]], crows_v.at[i, pl.ds(0, x_hbm.shape[1])],
            sem,
        )
        for i in range(NBC)
    ]


def _gather_body(ct_hbm, x_hbm, big_hbm, idx_a, idx_b, crows_a, crows_b,
                 sem_a, sem_b):
    wid = lax.axis_index("s") * NC + lax.axis_index("c")
    nb_per_w = x_hbm.shape[0] // NW
    b0 = wid * nb_per_w
    n_pairs = nb_per_w // (2 * NBC)

    def body(p, carry):
        bja = b0 + p * 2 * NBC
        bjb = bja + NBC
        starts_a = _chunk(ct_hbm, x_hbm, big_hbm, idx_a, crows_a, sem_a, bja)
        starts_b = _chunk(ct_hbm, x_hbm, big_hbm, idx_b, crows_b, sem_b, bjb)
        for s in starts_a:
            s.wait()
        pltpu.sync_copy(crows_a, big_hbm.at[pl.ds(bja, NBC)])
        for s in starts_b:
            s.wait()
        pltpu.sync_copy(crows_b, big_hbm.at[pl.ds(bjb, NBC)])
        return carry

    lax.fori_loop(0, n_pairs, body, 0)


def _sc_gather2(ct, x):
    b, l = x.shape
    mesh = plsc.VectorSubcoreMesh(core_axis_name="c", subcore_axis_name="s")
    k = pl.kernel(
        _gather_body,
        out_type=jax.ShapeDtypeStruct((b, LPAD, 2 * EMB), jnp.float32),
        mesh=mesh,
        scratch_types=[
            pltpu.VMEM((NBC, l), jnp.int32),
            pltpu.VMEM((NBC, l), jnp.int32),
            pltpu.VMEM((NBC, LPAD, 2 * EMB), jnp.float32),
            pltpu.VMEM((NBC, LPAD, 2 * EMB), jnp.float32),
            pltpu.SemaphoreType.DMA,
            pltpu.SemaphoreType.DMA,
        ],
        compiler_params=pltpu.CompilerParams(use_tc_tiling_on_sc=False),
    )
    return k(ct, x)


NBF = 256


def _fin_body(big_ref, ht_ref, lt_ref):
    v = big_ref[...]
    l = ht_ref.shape[2]
    ht_ref[...] = jnp.transpose(v[:, :l, :EMB], (0, 2, 1))
    lt_ref[...] = jnp.transpose(v[:, :l, EMB:], (0, 2, 1))


def _tc_finish(big, l):
    b = big.shape[0]
    return pl.pallas_call(
        _fin_body,
        grid=(b // NBF,),
        in_specs=[pl.BlockSpec((NBF, LPAD, 2 * EMB), lambda i: (i, 0, 0))],
        out_specs=[
            pl.BlockSpec((NBF, EMB, l), lambda i: (i, 0, 0)),
            pl.BlockSpec((NBF, EMB, l), lambda i: (i, 0, 0)),
        ],
        out_shape=[
            jax.ShapeDtypeStruct((b, EMB, l), jnp.float32),
            jax.ShapeDtypeStruct((b, EMB, l), jnp.float32),
        ],
    )(big)


def kernel(x, embed_table, router_w):
    l = x.shape[1]
    wct = jnp.concatenate(
        [jnp.eye(EMB, dtype=jnp.float32), router_w.T], axis=1
    )
    ct = _tc_combined_table(embed_table.T, wct)
    big = _sc_gather2(ct, x.astype(jnp.int32))
    ht, lt = _tc_finish(big, l)
    return (jnp.transpose(ht, (0, 2, 1)), jnp.transpose(lt, (0, 2, 1)))


# speedup vs baseline: 1.2643x; 1.0311x over previous
import jax
import jax.numpy as jnp
from jax import lax
from jax.experimental import pallas as pl
from jax.experimental.pallas import tpu as pltpu
from jax.experimental.pallas import tpu_sc as plsc

EMB = 64
NC, NS = 2, 16
NW = NC * NS
CT_BLK = 32768
NBC = 8
LPAD = 56


def _ct_body(tt_ref, w_ref, ct_ref):
    ct_ref[...] = lax.dot_general(
        tt_ref[...], w_ref[...], (((0,), (0,)), ((), ())),
        preferred_element_type=jnp.float32,
    )


def _tc_combined_table(table_t, wct):
    v = table_t.shape[1]
    nblk = -(-v // CT_BLK)
    return pl.pallas_call(
        _ct_body,
        grid=(nblk,),
        in_specs=[
            pl.BlockSpec((EMB, CT_BLK), lambda i: (0, i)),
            pl.BlockSpec((EMB, 2 * EMB), lambda i: (0, 0)),
        ],
        out_specs=pl.BlockSpec((CT_BLK, 2 * EMB), lambda i: (i, 0)),
        out_shape=jax.ShapeDtypeStruct((nblk * CT_BLK, 2 * EMB), jnp.float32),
    )(table_t, wct)


def _chunk(ct_hbm, x_hbm, big_hbm, idx_v, crows_v, sem, bj):
    pltpu.sync_copy(x_hbm.at[pl.ds(bj, NBC)], idx_v)
    return [
        pltpu.async_copy(
            ct_hbm.at[idx_v.at[i]], crows_v.at[i, pl.ds(0, x_hbm.shape[1])],
            sem,
        )
        for i in range(NBC)
    ]


def _gather_body(ct_hbm, x_hbm, big_hbm, idx_a, idx_b, crows_a, crows_b,
                 sem_a, sem_b):
    wid = lax.axis_index("s") * NC + lax.axis_index("c")
    nb_per_w = x_hbm.shape[0] // NW
    b0 = wid * nb_per_w
    n_pairs = nb_per_w // (2 * NBC)

    def body(p, carry):
        bja = b0 + p * 2 * NBC
        bjb = bja + NBC
        starts_a = _chunk(ct_hbm, x_hbm, big_hbm, idx_a, crows_a, sem_a, bja)
        starts_b = _chunk(ct_hbm, x_hbm, big_hbm, idx_b, crows_b, sem_b, bjb)
        for s in starts_a:
            s.wait()
        pltpu.sync_copy(crows_a, big_hbm.at[pl.ds(bja, NBC)])
        for s in starts_b:
            s.wait()
        pltpu.sync_copy(crows_b, big_hbm.at[pl.ds(bjb, NBC)])
        return carry

    lax.fori_loop(0, n_pairs, body, 0)


def _sc_gather2(ct, x):
    b, l = x.shape
    mesh = plsc.VectorSubcoreMesh(core_axis_name="c", subcore_axis_name="s")
    k = pl.kernel(
        _gather_body,
        out_type=jax.ShapeDtypeStruct((b, LPAD, 2 * EMB), jnp.float32),
        mesh=mesh,
        scratch_types=[
            pltpu.VMEM((NBC, l), jnp.int32),
            pltpu.VMEM((NBC, l), jnp.int32),
            pltpu.VMEM((NBC, LPAD, 2 * EMB), jnp.float32),
            pltpu.VMEM((NBC, LPAD, 2 * EMB), jnp.float32),
            pltpu.SemaphoreType.DMA,
            pltpu.SemaphoreType.DMA,
        ],
        compiler_params=pltpu.CompilerParams(use_tc_tiling_on_sc=False),
    )
    return k(ct, x)


NBF = 256


def _fin_body(big_ref, ht_ref, lt_ref):
    v = big_ref[...]
    l = ht_ref.shape[2]
    ht_ref[...] = jnp.transpose(v[:, :l, :EMB], (0, 2, 1))
    lt_ref[...] = v[:, :l, EMB:]


def _tc_finish(big, l):
    b = big.shape[0]
    return pl.pallas_call(
        _fin_body,
        grid=(b // NBF,),
        in_specs=[pl.BlockSpec((NBF, LPAD, 2 * EMB), lambda i: (i, 0, 0))],
        out_specs=[
            pl.BlockSpec((NBF, EMB, l), lambda i: (i, 0, 0)),
            pl.BlockSpec((NBF, l, EMB), lambda i: (i, 0, 0)),
        ],
        out_shape=[
            jax.ShapeDtypeStruct((b, EMB, l), jnp.float32),
            jax.ShapeDtypeStruct((b, l, EMB), jnp.float32),
        ],
    )(big)


def kernel(x, embed_table, router_w):
    l = x.shape[1]
    wct = jnp.concatenate(
        [jnp.eye(EMB, dtype=jnp.float32), router_w.T], axis=1
    )
    ct = _tc_combined_table(embed_table.T, wct)
    big = _sc_gather2(ct, x.astype(jnp.int32))
    ht, lt = _tc_finish(big, l)
    return (jnp.transpose(ht, (0, 2, 1)), lt)
